# Initial kernel scaffold; baseline (speedup 1.0000x reference)
import functools

import jax
import jax.numpy as jnp
from jax import lax
from jax.experimental import pallas as pl
from jax.experimental.pallas import tpu as pltpu
from jax.experimental.pallas import tpu_sc as plsc

_N = 10000
_E = 320000
_H = 32

_NC = 2
_NS = 16
_NW = _NC * _NS
_CH = 128
_K = -(-_E // (_NW * _CH))
_EPAD = _NW * _K * _CH
_NPAD = 10240
_RPT = _NPAD // _NS

_mesh = plsc.VectorSubcoreMesh(core_axis_name="c", subcore_axis_name="s")


@functools.partial(
    pl.kernel,
    out_type=jax.ShapeDtypeStruct((_NC, _NPAD), jnp.float32),
    mesh=_mesh,
    scratch_types=[
        pltpu.VMEM((_K, _CH), jnp.int32),
        pltpu.VMEM((_CH,), jnp.float32),
        pltpu.VMEM_SHARED((_NPAD,), jnp.float32),
    ],
)
def _sc_degree(dst_hbm, zero_hbm, out_hbm, dst_v, ones_v, acc_sh):
    c = lax.axis_index("c")
    s = lax.axis_index("s")
    wid = s * _NC + c
    pltpu.sync_copy(dst_hbm.at[wid], dst_v)
    for i in range(_CH // 16):
        ones_v[pl.ds(i * 16, 16)] = jnp.ones((16,), jnp.float32)
    pltpu.sync_copy(zero_hbm.at[pl.ds(s * _RPT, _RPT)],
                    acc_sh.at[pl.ds(s * _RPT, _RPT)])
    plsc.subcore_barrier()

    def body(j, carry):
        pltpu.sync_copy(ones_v, acc_sh.at[dst_v.at[j]], add=True)
        return carry

    lax.fori_loop(0, _K, body, 0)
    plsc.subcore_barrier()
    pltpu.sync_copy(acc_sh.at[pl.ds(s * _RPT, _RPT)],
                    out_hbm.at[c, pl.ds(s * _RPT, _RPT)])


@functools.partial(
    pl.kernel,
    out_type=jax.ShapeDtypeStruct((_NC, _NPAD, _H), jnp.float32),
    mesh=_mesh,
    scratch_types=[
        pltpu.VMEM((_K, _CH), jnp.int32),
        pltpu.VMEM((_K, _CH), jnp.int32),
        pltpu.VMEM((_CH, _H), jnp.float32),
        pltpu.VMEM_SHARED((_NPAD, _H), jnp.float32),
        pltpu.SemaphoreType.DMA,
    ],
)
def _sc_scatter(hs_hbm, src_hbm, dst_hbm, zero_hbm, out_hbm,
                src_v, dst_v, rows_v, acc_sh, gsem):
    c = lax.axis_index("c")
    s = lax.axis_index("s")
    wid = s * _NC + c
    pltpu.sync_copy(src_hbm.at[wid], src_v)
    pltpu.sync_copy(dst_hbm.at[wid], dst_v)
    pltpu.sync_copy(zero_hbm.at[pl.ds(s * _RPT, _RPT)],
                    acc_sh.at[pl.ds(s * _RPT, _RPT)])
    plsc.subcore_barrier()

    def body(j, carry):
        pltpu.async_copy(hs_hbm.at[src_v.at[j]], rows_v, gsem).wait()
        pltpu.sync_copy(rows_v, acc_sh.at[dst_v.at[j]], add=True)
        return carry

    lax.fori_loop(0, _K, body, 0)
    plsc.subcore_barrier()
    pltpu.sync_copy(acc_sh.at[pl.ds(s * _RPT, _RPT)],
                    out_hbm.at[c, pl.ds(s * _RPT, _RPT)])


def _dinv_body(c0_ref, c1_ref, o_ref):
    deg = c0_ref[...] + c1_ref[...] + 1.0
    o_ref[...] = lax.rsqrt(deg)


def _dense1_body(x_ref, w_ref, dinv_ref, o_ref):
    hw = jnp.dot(x_ref[...], w_ref[...], preferred_element_type=jnp.float32)
    o_ref[...] = dinv_ref[...] * hw


def _combine_body(p0_ref, p1_ref, hs_ref, dinv_ref, b_ref, w_ref, o_ref):
    agg = p0_ref[...] + p1_ref[...] + hs_ref[...]
    h = jnp.maximum(dinv_ref[...] * agg + b_ref[...], 0.0)
    o_ref[...] = dinv_ref[...] * jnp.dot(
        h, w_ref[...], preferred_element_type=jnp.float32)


def _final_body(p0_ref, p1_ref, hs_ref, dinv_ref, b_ref, o_ref):
    h = dinv_ref[...] * (p0_ref[...] + p1_ref[...] + hs_ref[...]) + b_ref[...]
    m = jnp.max(h, axis=1, keepdims=True)
    e = jnp.exp(h - m)
    lse = jnp.log(jnp.sum(e, axis=1, keepdims=True)) + m
    o_ref[...] = h - lse


def _tc(body, out_shape, *ins):
    return pl.pallas_call(
        body, out_shape=jax.ShapeDtypeStruct(out_shape, jnp.float32))(*ins)


def kernel(x, edge_index, W1, b1, W2, b2, W3, b3):
    src = edge_index[0]
    dst = edge_index[1]
    pad = _EPAD - _E
    src_p = jnp.concatenate(
        [src, jnp.zeros((pad,), jnp.int32)]).reshape(_NW, _K, _CH)
    dst_p = jnp.concatenate(
        [dst, jnp.full((pad,), _N, jnp.int32)]).reshape(_NW, _K, _CH)
    zero1 = jnp.zeros((_NPAD,), jnp.float32)
    zero2 = jnp.zeros((_NPAD, _H), jnp.float32)

    cnt = _sc_degree(dst_p, zero1)
    dinv2d = _tc(_dinv_body, (_NPAD // 128, 128),
                 cnt[0].reshape(_NPAD // 128, 128),
                 cnt[1].reshape(_NPAD // 128, 128))
    dinv = dinv2d.reshape(_NPAD)[:_N, None]

    hs1 = _tc(_dense1_body, (_N, _H), x, W1, dinv)
    p = _sc_scatter(hs1, src_p, dst_p, zero2)
    hs2 = _tc(_combine_body, (_N, _H),
              p[0, :_N], p[1, :_N], hs1, dinv, b1[None, :], W2)
    p = _sc_scatter(hs2, src_p, dst_p, zero2)
    hs3 = _tc(_combine_body, (_N, _H),
              p[0, :_N], p[1, :_N], hs2, dinv, b2[None, :], W3)
    p = _sc_scatter(hs3, src_p, dst_p, zero2)
    return _tc(_final_body, (_N, _H),
               p[0, :_N], p[1, :_N], hs3, dinv, b3[None, :])


# speedup vs baseline: 23.6376x; 23.6376x over previous
import functools

import jax
import jax.numpy as jnp
from jax import lax
from jax.experimental import pallas as pl
from jax.experimental.pallas import tpu as pltpu
from jax.experimental.pallas import tpu_sc as plsc

_N = 10000
_E = 320000
_H = 32

_NC = 2
_NS = 16
_NW = _NC * _NS
_CH = 128
_K = -(-_E // (_NW * _CH))
_EPAD = _NW * _K * _CH
_NPAD = 10240
_RPT = _NPAD // _NS

_mesh = plsc.VectorSubcoreMesh(core_axis_name="c", subcore_axis_name="s")


@functools.partial(
    pl.kernel,
    out_type=jax.ShapeDtypeStruct((_NC, _NPAD), jnp.float32),
    mesh=_mesh,
    scratch_types=[
        pltpu.VMEM((_K, _CH), jnp.int32),
        pltpu.VMEM((_CH,), jnp.float32),
        pltpu.VMEM_SHARED((_NPAD,), jnp.float32),
    ],
)
def _sc_degree(dst_hbm, zero_hbm, out_hbm, dst_v, ones_v, acc_sh):
    c = lax.axis_index("c")
    s = lax.axis_index("s")
    wid = s * _NC + c
    pltpu.sync_copy(dst_hbm.at[wid], dst_v)
    for i in range(_CH // 16):
        ones_v[pl.ds(i * 16, 16)] = jnp.ones((16,), jnp.float32)
    pltpu.sync_copy(zero_hbm.at[pl.ds(s * _RPT, _RPT)],
                    acc_sh.at[pl.ds(s * _RPT, _RPT)])
    plsc.subcore_barrier()

    def body(j, carry):
        pltpu.sync_copy(ones_v, acc_sh.at[dst_v.at[j]], add=True)
        return carry

    lax.fori_loop(0, _K, body, 0)
    plsc.subcore_barrier()
    pltpu.sync_copy(acc_sh.at[pl.ds(s * _RPT, _RPT)],
                    out_hbm.at[c, pl.ds(s * _RPT, _RPT)])


@functools.partial(
    pl.kernel,
    out_type=jax.ShapeDtypeStruct((_NC, _NPAD, _H), jnp.float32),
    mesh=_mesh,
    scratch_types=[
        pltpu.VMEM((_K, _CH), jnp.int32),
        pltpu.VMEM((_K, _CH), jnp.int32),
        pltpu.VMEM((_CH, _H), jnp.float32),
        pltpu.VMEM_SHARED((_NPAD, _H), jnp.float32),
        pltpu.SemaphoreType.DMA,
    ],
    compiler_params=pltpu.CompilerParams(use_tc_tiling_on_sc=False),
)
def _sc_scatter(hs_hbm, src_hbm, dst_hbm, zero_hbm, out_hbm,
                src_v, dst_v, rows_v, acc_sh, gsem):
    c = lax.axis_index("c")
    s = lax.axis_index("s")
    wid = s * _NC + c
    pltpu.sync_copy(src_hbm.at[wid], src_v)
    pltpu.sync_copy(dst_hbm.at[wid], dst_v)
    pltpu.sync_copy(zero_hbm.at[pl.ds(s * _RPT, _RPT)],
                    acc_sh.at[pl.ds(s * _RPT, _RPT)])
    plsc.subcore_barrier()

    def body(j, carry):
        pltpu.async_copy(hs_hbm.at[src_v.at[j]], rows_v, gsem).wait()
        pltpu.sync_copy(rows_v, acc_sh.at[dst_v.at[j]], add=True)
        return carry

    lax.fori_loop(0, _K, body, 0)
    plsc.subcore_barrier()
    pltpu.sync_copy(acc_sh.at[pl.ds(s * _RPT, _RPT)],
                    out_hbm.at[c, pl.ds(s * _RPT, _RPT)])


def _dinv_body(c0_ref, c1_ref, o_ref):
    deg = c0_ref[...] + c1_ref[...] + 1.0
    o_ref[...] = lax.rsqrt(deg)


def _dense1_body(x_ref, w_ref, dinv_ref, o_ref):
    hw = jnp.dot(x_ref[...], w_ref[...], preferred_element_type=jnp.float32)
    o_ref[...] = dinv_ref[...] * hw


def _combine_body(p0_ref, p1_ref, hs_ref, dinv_ref, b_ref, w_ref, o_ref):
    agg = p0_ref[...] + p1_ref[...] + hs_ref[...]
    h = jnp.maximum(dinv_ref[...] * agg + b_ref[...], 0.0)
    o_ref[...] = dinv_ref[...] * jnp.dot(
        h, w_ref[...], preferred_element_type=jnp.float32)


def _final_body(p0_ref, p1_ref, hs_ref, dinv_ref, b_ref, o_ref):
    h = dinv_ref[...] * (p0_ref[...] + p1_ref[...] + hs_ref[...]) + b_ref[...]
    m = jnp.max(h, axis=1, keepdims=True)
    e = jnp.exp(h - m)
    lse = jnp.log(jnp.sum(e, axis=1, keepdims=True)) + m
    o_ref[...] = h - lse


def _tc(body, out_shape, *ins):
    return pl.pallas_call(
        body, out_shape=jax.ShapeDtypeStruct(out_shape, jnp.float32))(*ins)


def kernel(x, edge_index, W1, b1, W2, b2, W3, b3):
    src = edge_index[0]
    dst = edge_index[1]
    pad = _EPAD - _E
    src_p = jnp.concatenate(
        [src, jnp.zeros((pad,), jnp.int32)]).reshape(_NW, _K, _CH)
    dst_p = jnp.concatenate(
        [dst, jnp.full((pad,), _N, jnp.int32)]).reshape(_NW, _K, _CH)
    zero1 = jnp.zeros((_NPAD,), jnp.float32)
    zero2 = jnp.zeros((_NPAD, _H), jnp.float32)

    cnt = _sc_degree(dst_p, zero1)
    dinv2d = _tc(_dinv_body, (_NPAD // 128, 128),
                 cnt[0].reshape(_NPAD // 128, 128),
                 cnt[1].reshape(_NPAD // 128, 128))
    dinv = dinv2d.reshape(_NPAD)[:_N, None]

    hs1 = _tc(_dense1_body, (_N, _H), x, W1, dinv)
    p = _sc_scatter(hs1, src_p, dst_p, zero2)
    hs2 = _tc(_combine_body, (_N, _H),
              p[0, :_N], p[1, :_N], hs1, dinv, b1[None, :], W2)
    p = _sc_scatter(hs2, src_p, dst_p, zero2)
    hs3 = _tc(_combine_body, (_N, _H),
              p[0, :_N], p[1, :_N], hs2, dinv, b2[None, :], W3)
    p = _sc_scatter(hs3, src_p, dst_p, zero2)
    return _tc(_final_body, (_N, _H),
               p[0, :_N], p[1, :_N], hs3, dinv, b3[None, :])
